# baseline (speedup 1.0000x reference)
import functools

import jax
import jax.numpy as jnp
from jax import lax
from jax.experimental import pallas as pl
from jax.experimental.pallas import tpu as pltpu
from jax.experimental.pallas import tpu_sc as plsc

NBUF = 4
LEAD = 2


def kernel(focuses, mask, pe):
    B, L = focuses.shape
    V, H = pe.shape
    N = B * L
    info = plsc.get_sparse_core_info()
    nc, ns = info.num_cores, info.num_subcores
    nw = nc * ns
    nb = B // nw
    per_w = nb * L
    n_outer = nb // NBUF
    assert nb * nw == B and n_outer * NBUF == nb
    C0 = 128
    C1 = L - C0
    assert 0 < C1 <= 128 and C0 % 8 == 0
    fix_iters = per_w // 16
    assert per_w % 16 == 0

    zero_row = V
    pe_ext = jnp.concatenate([pe, jnp.zeros((1, H), jnp.float32)], axis=0)
    foc2 = focuses.reshape(nw, per_w)
    msk2 = mask.astype(jnp.int32).reshape(nw, per_w)

    @functools.partial(
        pl.kernel,
        mesh=plsc.VectorSubcoreMesh(core_axis_name="c", subcore_axis_name="s"),
        compiler_params=pltpu.CompilerParams(use_tc_tiling_on_sc=False),
        out_type=jax.ShapeDtypeStruct((B, L, H), jnp.float32),
        scratch_types=[
            pltpu.VMEM((per_w,), jnp.int32),
            pltpu.VMEM((per_w,), jnp.int32),
            pltpu.VMEM((NBUF, L, H), jnp.float32),
            pltpu.VMEM_SHARED((V + 1, H), jnp.float32),
        ]
        + [pltpu.SemaphoreType.DMA] * (2 * NBUF),
    )
    def fe_kernel(pe_hbm, foc_hbm, msk_hbm, out_hbm, idx_v, msk_v, rows_v, pe_sh, *sems):
        gsems = sems[:NBUF]
        wsems = sems[NBUF:]
        c = lax.axis_index("c")
        s = lax.axis_index("s")
        wid = s * nc + c
        b0 = wid * nb

        @pl.when(s == 0)
        def _():
            pltpu.sync_copy(pe_hbm, pe_sh)

        pltpu.sync_copy(foc_hbm.at[wid], idx_v)
        pltpu.sync_copy(msk_hbm.at[wid], msk_v)

        def fix(t, carry):
            m = msk_v[pl.ds(t * 16, 16)]
            f = idx_v[pl.ds(t * 16, 16)]
            idx_v[pl.ds(t * 16, 16)] = jnp.where(m > 0, f, zero_row)
            return carry

        lax.fori_loop(0, fix_iters, fix, 0)
        plsc.subcore_barrier()

        def gathers(j, slot):
            pltpu.async_copy(
                pe_sh.at[idx_v.at[pl.ds(j * L, C0)]],
                rows_v.at[slot, pl.ds(0, C0)],
                gsems[slot],
            )
            pltpu.async_copy(
                pe_sh.at[idx_v.at[pl.ds(j * L + C0, C1)]],
                rows_v.at[slot, pl.ds(C0, C1)],
                gsems[slot],
            )

        def wait_gathers(j, slot):
            pltpu.make_async_copy(
                pe_sh.at[idx_v.at[pl.ds(j * L, C0)]],
                rows_v.at[slot, pl.ds(0, C0)],
                gsems[slot],
            ).wait()
            pltpu.make_async_copy(
                pe_sh.at[idx_v.at[pl.ds(j * L + C0, C1)]],
                rows_v.at[slot, pl.ds(C0, C1)],
                gsems[slot],
            ).wait()

        def outer(o, carry):
            for b in range(NBUF):
                j = o * NBUF + b

                @pl.when(j >= NBUF)
                def _():
                    pltpu.make_async_copy(
                        rows_v.at[b], out_hbm.at[b0 + j - NBUF], wsems[b]
                    ).wait()

                gathers(j, b)

                @pl.when(j >= LEAD)
                def _():
                    b2 = (b - LEAD) % NBUF
                    j2 = j - LEAD
                    wait_gathers(j2, b2)
                    pltpu.async_copy(rows_v.at[b2], out_hbm.at[b0 + j2], wsems[b2])

            return carry

        lax.fori_loop(0, n_outer, outer, 0)

        for t in range(LEAD):
            j2 = nb - LEAD + t
            b2 = j2 % NBUF
            wait_gathers(j2, b2)
            pltpu.async_copy(rows_v.at[b2], out_hbm.at[b0 + j2], wsems[b2])
        for b in range(NBUF):
            j2 = nb - NBUF + b
            pltpu.make_async_copy(
                rows_v.at[b], out_hbm.at[b0 + j2], wsems[b]
            ).wait()

    return fe_kernel(pe_ext, foc2, msk2)
